# baseline (speedup 1.0000x reference)
import jax
import jax.numpy as jnp
from jax import lax
from jax.experimental import pallas as pl
from jax.experimental.pallas import tpu as pltpu
from jax.experimental.pallas import tpu_sc as plsc

BATCH = 16384
NUM_FEATURES = 5
EMBED_DIM = 32
NC, NS, LANES = 2, 16, 16
NW = NC * NS
BPW = BATCH // NW
NCHUNK = BPW // LANES
NGROUP = 4


def _sc_body(idx_hbm, tableT_hbm, out_hbm, idx_v, tiles, vals, *sems):
    wid = lax.axis_index("s") * NC + lax.axis_index("c")
    base = wid * BPW

    pltpu.sync_copy(idx_hbm.at[pl.ds(base, BPW)], idx_v)

    c_lo = lax.iota(jnp.int32, LANES)

    def lane_scalar(vec, b):
        return jnp.max(jnp.where(c_lo == b, vec, 0))

    def fetch(r, b):
        rb = pl.multiple_of((r >> 7) << 7, 128)
        for a in range(NGROUP):
            pltpu.make_async_copy(
                tableT_hbm.at[pl.ds(8 * a, 8), pl.ds(rb, 128)],
                tiles.at[b].at[pl.ds(8 * a, 8)],
                sems[b],
            ).start()

    vec0 = idx_v[pl.ds(0, LANES)]
    for b in range(LANES):
        fetch(lane_scalar(vec0, b), b)

    a_lo = c_lo >> 3
    cc_lo = c_lo & 7

    def chunk(kk, vec_cur):
        nxt = jnp.minimum(kk + 1, NCHUNK - 1)
        vec_next = idx_v[pl.ds(nxt * LANES, LANES)]
        for b in range(LANES):
            k = kk * LANES + b
            pltpu.make_async_copy(
                tableT_hbm.at[pl.ds(0, 32), pl.ds(0, 128)],
                tiles.at[b],
                sems[b],
            ).wait()
            r = lane_scalar(vec_cur, b)
            j = jnp.full((LANES,), 1, jnp.int32) * (r & 127)
            v0 = plsc.load_gather(tiles.at[b], [c_lo, j])
            v1 = plsc.load_gather(tiles.at[b], [c_lo + LANES, j])
            vals[pl.ds(k * EMBED_DIM, LANES)] = v0
            vals[pl.ds(k * EMBED_DIM + LANES, LANES)] = v1

            @pl.when(kk < NCHUNK - 1)
            def _():
                fetch(lane_scalar(vec_next, b), b)

        return vec_next

    lax.fori_loop(0, NCHUNK, chunk, vec0)

    pltpu.sync_copy(
        vals, out_hbm.at[pl.ds(base * EMBED_DIM, BPW * EMBED_DIM)]
    )


@jax.jit
def kernel(x, table):
    idx = x[:, NUM_FEATURES].astype(jnp.int32)
    tableT = table.T
    mesh = plsc.VectorSubcoreMesh(core_axis_name="c", subcore_axis_name="s")
    run = pl.kernel(
        _sc_body,
        out_type=jax.ShapeDtypeStruct((BATCH * EMBED_DIM,), jnp.float32),
        mesh=mesh,
        compiler_params=pltpu.CompilerParams(needs_layout_passes=False),
        scratch_types=[
            pltpu.VMEM((BPW,), jnp.int32),
            pltpu.VMEM((LANES, EMBED_DIM, 128), jnp.float32),
            pltpu.VMEM((BPW * EMBED_DIM,), jnp.float32),
        ]
        + [pltpu.SemaphoreType.DMA] * LANES,
    )
    flat = run(idx, tableT)
    emb = flat.reshape(BATCH, EMBED_DIM)
    return jnp.concatenate([x[:, :NUM_FEATURES], emb], axis=1)


# speedup vs baseline: 1.0008x; 1.0008x over previous
import jax
import jax.numpy as jnp
from jax import lax
from jax.experimental import pallas as pl
from jax.experimental.pallas import tpu as pltpu
from jax.experimental.pallas import tpu_sc as plsc

BATCH = 16384
NUM_FEATURES = 5
EMBED_DIM = 32
NC, NS, LANES = 2, 16, 16
NW = NC * NS
BPW = BATCH // NW
NCHUNK = BPW // LANES


def _sc_body(idx_hbm, tableT_hbm, out_hbm, idx_v, tiles, vals, *sems):
    wid = lax.axis_index("s") * NC + lax.axis_index("c")
    base = wid * BPW

    pltpu.sync_copy(idx_hbm.at[pl.ds(base, BPW)], idx_v)

    c_lo = lax.iota(jnp.int32, LANES)

    def lane_scalar(vec, b):
        return jnp.max(jnp.where(c_lo == b, vec, 0))

    def fetch(r, b):
        rb = pl.multiple_of((r >> 7) << 7, 128)
        pltpu.make_async_copy(
            tableT_hbm.at[:, pl.ds(rb, 128)], tiles.at[b], sems[b]
        ).start()

    vec0 = idx_v[pl.ds(0, LANES)]
    for b in range(LANES):
        fetch(lane_scalar(vec0, b), b)

    def chunk(kk, vec_cur):
        nxt = jnp.minimum(kk + 1, NCHUNK - 1)
        vec_next = idx_v[pl.ds(nxt * LANES, LANES)]
        for b in range(LANES):
            k = kk * LANES + b
            pltpu.make_async_copy(
                tableT_hbm.at[:, pl.ds(0, 128)], tiles.at[b], sems[b]
            ).wait()
            r = lane_scalar(vec_cur, b)
            j = jnp.full((LANES,), 1, jnp.int32) * (r & 127)
            v0 = plsc.load_gather(tiles.at[b], [c_lo, j])
            v1 = plsc.load_gather(tiles.at[b], [c_lo + LANES, j])
            vals[pl.ds(k * EMBED_DIM, LANES)] = v0
            vals[pl.ds(k * EMBED_DIM + LANES, LANES)] = v1

            @pl.when(kk < NCHUNK - 1)
            def _():
                fetch(lane_scalar(vec_next, b), b)

        return vec_next

    lax.fori_loop(0, NCHUNK, chunk, vec0)

    pltpu.sync_copy(
        vals, out_hbm.at[pl.ds(base * EMBED_DIM, BPW * EMBED_DIM)]
    )


@jax.jit
def kernel(x, table):
    idx = x[:, NUM_FEATURES].astype(jnp.int32)
    tableT = table.T
    mesh = plsc.VectorSubcoreMesh(core_axis_name="c", subcore_axis_name="s")
    run = pl.kernel(
        _sc_body,
        out_type=jax.ShapeDtypeStruct((BATCH * EMBED_DIM,), jnp.float32),
        mesh=mesh,
        compiler_params=pltpu.CompilerParams(needs_layout_passes=False),
        scratch_types=[
            pltpu.VMEM((BPW,), jnp.int32),
            pltpu.VMEM((LANES, EMBED_DIM, 128), jnp.float32),
            pltpu.VMEM((BPW * EMBED_DIM,), jnp.float32),
        ]
        + [pltpu.SemaphoreType.DMA] * LANES,
    )
    flat = run(idx, tableT)
    emb = flat.reshape(BATCH, EMBED_DIM)
    return jnp.concatenate([x[:, :NUM_FEATURES], emb], axis=1)
